# baseline (speedup 1.0000x reference)
import functools

import jax
import jax.numpy as jnp
from jax import lax
from jax.experimental import pallas as pl
from jax.experimental.pallas import tpu as pltpu
from jax.experimental.pallas import tpu_sc as plsc

N_NODES = 10000
N_EDGES = 320000
D = 128
DH = 64
D_OUT = 64

NC = 2
NS = 16
EPT = N_EDGES // NS
CHUNK = 80
NCHUNK = EPT // CHUNK
NBUF = 5
GAHEAD = 3
RPT = 640
ZROWS = 80


def _spmm2x_body(xa_hbm, xb_hbm, src_hbm, dst_hbm, ew_hbm,
                 z1a_hbm, z1b_hbm, z2a_hbm, z2b_hbm,
                 src_b, dst_b, ew_b, rows_v, zbuf_v, acc1, acc2,
                 isem, gsem, ssem, osem):
    cid = lax.axis_index("c")
    sid = lax.axis_index("s")
    row0 = sid * RPT
    nrows_last = N_NODES - (NS - 1) * RPT

    def zrow(r, carry):
        for c in range(DH // 16):
            zbuf_v[r, pl.ds(c * 16, 16)] = jnp.zeros((16,), jnp.float32)
        return carry
    lax.fori_loop(0, ZROWS, zrow, 0)

    for a in (acc1, acc2):
        @pl.when(sid < NS - 1)
        def _zero_full():
            for j in range(RPT // ZROWS):
                pltpu.sync_copy(zbuf_v, a.at[pl.ds(row0 + j * ZROWS, ZROWS)])

        @pl.when(sid == NS - 1)
        def _zero_last():
            for j in range(nrows_last // ZROWS):
                pltpu.sync_copy(zbuf_v, a.at[pl.ds(row0 + j * ZROWS, ZROWS)])

    plsc.subcore_barrier()

    inv_n = jnp.float32(1.0 / N_NODES)

    def _issue_idx(ci, b):
        pltpu.async_copy(src_hbm.at[sid, ci], src_b[b], isem[b])
        pltpu.async_copy(dst_hbm.at[sid, ci], dst_b[b], isem[b])
        pltpu.async_copy(ew_hbm.at[sid, ci], ew_b[b], isem[b])

    def _wait_idx(ci, b):
        pltpu.make_async_copy(src_hbm.at[sid, ci], src_b[b], isem[b]).wait()
        pltpu.make_async_copy(dst_hbm.at[sid, ci], dst_b[b], isem[b]).wait()
        pltpu.make_async_copy(ew_hbm.at[sid, ci], ew_b[b], isem[b]).wait()

    def _run_pass(issue_gather, wait_gather, acc):
        def _wait_scatter(b):
            pltpu.make_async_copy(rows_v[b], acc.at[dst_b[b]],
                                  ssem[b]).wait()

        for j in range(NBUF):
            _issue_idx(j, j)
        for j in range(GAHEAD):
            _wait_idx(j, j)
            issue_gather(j)

        def ring_body(cg, carry):
            for b in range(NBUF):
                ci = cg * NBUF + b
                wait_gather(b)

                bg = (b + GAHEAD) % NBUF

                @pl.when(ci + GAHEAD < NCHUNK)
                def _launch_ahead():
                    _wait_idx(ci + GAHEAD, bg)

                    @pl.when(ci >= NBUF - GAHEAD)
                    def _reuse_guard():
                        _wait_scatter(bg)

                    issue_gather(bg)

                def group_body(g, inner):
                    ew16 = ew_b[b][pl.ds(g * 16, 16)] * inv_n
                    for j0 in range(0, 16, 4):
                        es = [g * 16 + j0 + t for t in range(4)]
                        ws = [ew16[j0 + t] for t in range(4)]
                        vals = [rows_v[b][es[t], pl.ds(c * 16, 16)]
                                for t in range(4) for c in range(DH // 16)]
                        for t in range(4):
                            for c in range(DH // 16):
                                rows_v[b][es[t], pl.ds(c * 16, 16)] = (
                                    vals[t * (DH // 16) + c] * ws[t])
                    return inner
                lax.fori_loop(0, CHUNK // 16, group_body, 0)

                pltpu.async_copy(rows_v[b], acc.at[dst_b[b]], ssem[b],
                                 add=True)

                @pl.when(ci + NBUF < NCHUNK)
                def _prefetch_idx():
                    _issue_idx(ci + NBUF, b)
            return carry
        lax.fori_loop(0, NCHUNK // NBUF, ring_body, 0)

        for b in range(NBUF):
            _wait_scatter(b)

    def g1_issue(b):
        @pl.when(cid == 0)
        def _ga():
            pltpu.async_copy(xa_hbm.at[src_b[b]], rows_v[b], gsem[b])

        @pl.when(cid == 1)
        def _gb():
            pltpu.async_copy(xb_hbm.at[src_b[b]], rows_v[b], gsem[b])

    def g1_wait(b):
        @pl.when(cid == 0)
        def _wa():
            pltpu.make_async_copy(xa_hbm.at[src_b[b]], rows_v[b],
                                  gsem[b]).wait()

        @pl.when(cid == 1)
        def _wb():
            pltpu.make_async_copy(xb_hbm.at[src_b[b]], rows_v[b],
                                  gsem[b]).wait()

    _run_pass(g1_issue, g1_wait, acc1)
    plsc.subcore_barrier()

    @pl.when(cid == 0)
    def _outa1():
        @pl.when(sid < NS - 1)
        def _full():
            pltpu.async_copy(acc1.at[pl.ds(row0, RPT)],
                             z1a_hbm.at[pl.ds(row0, RPT)], osem)

        @pl.when(sid == NS - 1)
        def _last():
            pltpu.async_copy(acc1.at[pl.ds(row0, nrows_last)],
                             z1a_hbm.at[pl.ds(row0, nrows_last)], osem)

    @pl.when(cid == 1)
    def _outb1():
        @pl.when(sid < NS - 1)
        def _full():
            pltpu.async_copy(acc1.at[pl.ds(row0, RPT)],
                             z1b_hbm.at[pl.ds(row0, RPT)], osem)

        @pl.when(sid == NS - 1)
        def _last():
            pltpu.async_copy(acc1.at[pl.ds(row0, nrows_last)],
                             z1b_hbm.at[pl.ds(row0, nrows_last)], osem)

    def g2_issue(b):
        pltpu.async_copy(acc1.at[src_b[b]], rows_v[b], gsem[b])

    def g2_wait(b):
        pltpu.make_async_copy(acc1.at[src_b[b]], rows_v[b], gsem[b]).wait()

    _run_pass(g2_issue, g2_wait, acc2)

    @pl.when(sid < NS - 1)
    def _drain_full():
        @pl.when(cid == 0)
        def _a():
            pltpu.make_async_copy(acc1.at[pl.ds(row0, RPT)],
                                  z1a_hbm.at[pl.ds(row0, RPT)], osem).wait()

        @pl.when(cid == 1)
        def _b():
            pltpu.make_async_copy(acc1.at[pl.ds(row0, RPT)],
                                  z1b_hbm.at[pl.ds(row0, RPT)], osem).wait()

    @pl.when(sid == NS - 1)
    def _drain_last():
        @pl.when(cid == 0)
        def _a():
            pltpu.make_async_copy(acc1.at[pl.ds(row0, nrows_last)],
                                  z1a_hbm.at[pl.ds(row0, nrows_last)],
                                  osem).wait()

        @pl.when(cid == 1)
        def _b():
            pltpu.make_async_copy(acc1.at[pl.ds(row0, nrows_last)],
                                  z1b_hbm.at[pl.ds(row0, nrows_last)],
                                  osem).wait()

    plsc.subcore_barrier()

    @pl.when(cid == 0)
    def _outa2():
        @pl.when(sid < NS - 1)
        def _full():
            pltpu.sync_copy(acc2.at[pl.ds(row0, RPT)],
                            z2a_hbm.at[pl.ds(row0, RPT)])

        @pl.when(sid == NS - 1)
        def _last():
            pltpu.sync_copy(acc2.at[pl.ds(row0, nrows_last)],
                            z2a_hbm.at[pl.ds(row0, nrows_last)])

    @pl.when(cid == 1)
    def _outb2():
        @pl.when(sid < NS - 1)
        def _full():
            pltpu.sync_copy(acc2.at[pl.ds(row0, RPT)],
                            z2b_hbm.at[pl.ds(row0, RPT)])

        @pl.when(sid == NS - 1)
        def _last():
            pltpu.sync_copy(acc2.at[pl.ds(row0, nrows_last)],
                            z2b_hbm.at[pl.ds(row0, nrows_last)])


def _spmm2x(xa, xb, src3, dst3, ew3):
    mesh = plsc.VectorSubcoreMesh(core_axis_name="c", subcore_axis_name="s")
    f = pl.kernel(
        _spmm2x_body,
        compiler_params=pltpu.CompilerParams(use_tc_tiling_on_sc=False),
        out_type=tuple(jax.ShapeDtypeStruct((N_NODES, DH), jnp.float32)
                       for _ in range(4)),
        mesh=mesh,
        scratch_types=[
            [pltpu.VMEM((CHUNK,), jnp.int32)] * NBUF,
            [pltpu.VMEM((CHUNK,), jnp.int32)] * NBUF,
            [pltpu.VMEM((CHUNK,), jnp.float32)] * NBUF,
            [pltpu.VMEM((CHUNK, DH), jnp.float32)] * NBUF,
            pltpu.VMEM((ZROWS, DH), jnp.float32),
            pltpu.VMEM_SHARED((N_NODES, DH), jnp.float32),
            pltpu.VMEM_SHARED((N_NODES, DH), jnp.float32),
            [pltpu.SemaphoreType.DMA] * NBUF,
            [pltpu.SemaphoreType.DMA] * NBUF,
            [pltpu.SemaphoreType.DMA] * NBUF,
            pltpu.SemaphoreType.DMA,
        ],
    )
    return f(xa, xb, src3, dst3, ew3)


BR = 1000


def _layer1_body(x_ref, za1_ref, zb1_ref, za2_ref, zb2_ref,
                 w00_ref, w01a_ref, w01b_ref, w02a_ref, w02b_ref,
                 ya_ref, yb_ref):
    def dot(a, b):
        return jnp.dot(a, b, preferred_element_type=jnp.float32)
    y = dot(x_ref[...], w00_ref[...])
    y += dot(za1_ref[...], w01a_ref[...]) + dot(zb1_ref[...], w01b_ref[...])
    y += dot(za2_ref[...], w02a_ref[...]) + dot(zb2_ref[...], w02b_ref[...])
    y = jnp.maximum(y, 0.0)
    ya_ref[...] = y[:, :DH]
    yb_ref[...] = y[:, DH:]


def _tc_layer1(x, za1, zb1, za2, zb2, W0):
    grid = (N_NODES // BR,)
    fblk = pl.BlockSpec((BR, D), lambda i: (i, 0))
    hblk = pl.BlockSpec((BR, DH), lambda i: (i, 0))
    wf = pl.BlockSpec((D, D), lambda i: (0, 0))
    wh = pl.BlockSpec((DH, D), lambda i: (0, 0))
    return pl.pallas_call(
        _layer1_body,
        grid=grid,
        in_specs=[fblk, hblk, hblk, hblk, hblk, wf, wh, wh, wh, wh],
        out_specs=[hblk, hblk],
        out_shape=[jax.ShapeDtypeStruct((N_NODES, DH), jnp.float32),
                   jax.ShapeDtypeStruct((N_NODES, DH), jnp.float32)],
    )(x, za1, zb1, za2, zb2,
      W0[0], W0[1, :DH], W0[1, DH:], W0[2, :DH], W0[2, DH:])


def _layer2_body(ya_ref, yb_ref, ha1_ref, hb1_ref, ha2_ref, hb2_ref,
                 w10_ref, w11a_ref, w11b_ref, w12a_ref, w12b_ref,
                 wm_ref, bm_ref, out_ref):
    def dot(a, b):
        return jnp.dot(a, b, preferred_element_type=jnp.float32)
    y1 = jnp.concatenate([ya_ref[...], yb_ref[...]], axis=1)
    y2 = dot(y1, w10_ref[...])
    y2 += dot(ha1_ref[...], w11a_ref[...]) + dot(hb1_ref[...], w11b_ref[...])
    y2 += dot(ha2_ref[...], w12a_ref[...]) + dot(hb2_ref[...], w12b_ref[...])
    y2 = jnp.maximum(y2, 0.0)
    out_ref[...] = dot(y2, wm_ref[...]) + bm_ref[...]


def _tc_layer2(ya, yb, ha1, hb1, ha2, hb2, W1, Wm, bm):
    grid = (N_NODES // BR,)
    hblk = pl.BlockSpec((BR, DH), lambda i: (i, 0))
    wf = pl.BlockSpec((D, D), lambda i: (0, 0))
    wh = pl.BlockSpec((DH, D), lambda i: (0, 0))
    return pl.pallas_call(
        _layer2_body,
        grid=grid,
        in_specs=[hblk, hblk, hblk, hblk, hblk, hblk, wf, wh, wh, wh, wh,
                  pl.BlockSpec((D, D_OUT), lambda i: (0, 0)),
                  pl.BlockSpec((1, D_OUT), lambda i: (0, 0))],
        out_specs=pl.BlockSpec((BR, D_OUT), lambda i: (i, 0)),
        out_shape=jax.ShapeDtypeStruct((N_NODES, D_OUT), jnp.float32),
    )(ya, yb, ha1, hb1, ha2, hb2,
      W1[0], W1[1, :DH], W1[1, DH:], W1[2, :DH], W1[2, DH:], Wm,
      bm.reshape(1, D_OUT))


def kernel(x, edge_index, edge_weight, W0, W1, Wm, bm):
    src3 = edge_index[1].astype(jnp.int32).reshape(NS, NCHUNK, CHUNK)
    dst3 = edge_index[0].astype(jnp.int32).reshape(NS, NCHUNK, CHUNK)
    ew3 = edge_weight.reshape(NS, NCHUNK, CHUNK)
    xa, xb = x[:, :DH], x[:, DH:]

    za1, zb1, za2, zb2 = _spmm2x(xa, xb, src3, dst3, ew3)
    ya, yb = _tc_layer1(x, za1, zb1, za2, zb2, W0)
    ha1, hb1, ha2, hb2 = _spmm2x(ya, yb, src3, dst3, ew3)
    return _tc_layer2(ya, yb, ha1, hb1, ha2, hb2, W1, Wm, bm)
